# baseline (speedup 1.0000x reference)
import functools

import jax
import jax.numpy as jnp
from jax import lax
from jax.experimental import pallas as pl
from jax.experimental.pallas import tpu as pltpu, tpu_sc as plsc

_E = 8
_R = 8
_SCALING = 32.0 / 8.0
_BT = 2048

_NC = 2
_NS = 16
_NW = _NC * _NS


def _pass1_body(x_ref, wgt_ref, bg_ref, af_ref, lt_ref, h_ref):
    xv = x_ref[...]
    logits = jnp.dot(xv, wgt_ref[...], preferred_element_type=jnp.float32)
    lt_ref[...] = logits.T + bg_ref[...]
    h_ref[...] = jnp.dot(
        xv.astype(jnp.bfloat16), af_ref[...], preferred_element_type=jnp.float32
    ).astype(jnp.bfloat16)


def _pass3_body(h_ref, wt_ref, rep_ref, bf_ref, o_ref):
    w_rep = lax.dot_general(
        wt_ref[...],
        rep_ref[...],
        dimension_numbers=(((0,), (0,)), ((), ())),
        preferred_element_type=jnp.float32,
    )
    g = (h_ref[...].astype(jnp.float32) * w_rep).astype(jnp.bfloat16)
    o_ref[...] = jnp.dot(g, bf_ref[...], preferred_element_type=jnp.float32)


def _sc_router(T):
    tok_per_w = T // _NW
    chunk = 16
    n_steps = tok_per_w // chunk

    mesh = plsc.VectorSubcoreMesh(core_axis_name="c", subcore_axis_name="s")

    @functools.partial(
        pl.kernel,
        out_type=jax.ShapeDtypeStruct((_E, T), jnp.float32),
        mesh=mesh,
        scratch_types=[
            pltpu.VMEM((_E, tok_per_w), jnp.float32),
            pltpu.VMEM((_E, tok_per_w), jnp.float32),
        ],
    )
    def k(lt_hbm, w_hbm, lg_v, w_v):
        wid = lax.axis_index("s") * _NC + lax.axis_index("c")
        base = wid * tok_per_w
        pltpu.sync_copy(lt_hbm.at[:, pl.ds(base, tok_per_w)], lg_v)

        def step(c, carry):
            sl = pl.ds(c * chunk, chunk)
            le = [lg_v[e, sl] for e in range(_E)]
            m1 = le[0]
            for e in range(1, _E):
                m1 = jnp.maximum(m1, le[e])
            big = jnp.full((chunk,), _E, jnp.int32)
            i1 = big
            for e in range(_E):
                i1 = jnp.minimum(i1, jnp.where(le[e] == m1, jnp.int32(e), big))
            neg = jnp.float32(-jnp.inf)
            l2 = [jnp.where(i1 == e, neg, le[e]) for e in range(_E)]
            m2 = l2[0]
            for e in range(1, _E):
                m2 = jnp.maximum(m2, l2[e])
            i2 = big
            for e in range(_E):
                i2 = jnp.minimum(i2, jnp.where(l2[e] == m2, jnp.int32(e), big))
            p2 = jnp.exp(m2 - m1)
            w1 = 1.0 / (1.0 + p2)
            w2 = 1.0 - w1
            zero = jnp.zeros((chunk,), jnp.float32)
            for e in range(_E):
                w_v[e, sl] = jnp.where(i1 == e, w1, jnp.where(i2 == e, w2, zero))
            return carry

        lax.fori_loop(0, n_steps, step, jnp.int32(0), unroll=False)
        pltpu.sync_copy(w_v, w_hbm.at[:, pl.ds(base, tok_per_w)])

    return k


@jax.jit
def kernel(x, Wg, bg, A, B):
    T, D = x.shape
    E, R, _ = A.shape
    wgt = Wg.T
    a_flat = A.reshape(E * R, D).T.astype(jnp.bfloat16)
    b_flat = (
        (B.transpose(0, 2, 1) * jnp.float32(_SCALING))
        .reshape(E * R, D)
        .astype(jnp.bfloat16)
    )
    rep = jnp.repeat(jnp.eye(E, dtype=jnp.float32), R, axis=1)
    bg2 = bg.reshape(E, 1)

    grid = (T // _BT,)
    logits_t, h = pl.pallas_call(
        _pass1_body,
        grid=grid,
        in_specs=[
            pl.BlockSpec((_BT, D), lambda i: (i, 0)),
            pl.BlockSpec((D, E), lambda i: (0, 0)),
            pl.BlockSpec((E, 1), lambda i: (0, 0)),
            pl.BlockSpec((D, E * R), lambda i: (0, 0)),
        ],
        out_specs=[
            pl.BlockSpec((E, _BT), lambda i: (0, i)),
            pl.BlockSpec((_BT, E * R), lambda i: (i, 0)),
        ],
        out_shape=[
            jax.ShapeDtypeStruct((E, T), jnp.float32),
            jax.ShapeDtypeStruct((T, E * R), jnp.bfloat16),
        ],
    )(x, wgt, bg2, a_flat)

    w_t = _sc_router(T)(logits_t)

    out = pl.pallas_call(
        _pass3_body,
        grid=grid,
        in_specs=[
            pl.BlockSpec((_BT, E * R), lambda i: (i, 0)),
            pl.BlockSpec((E, _BT), lambda i: (0, i)),
            pl.BlockSpec((E, E * R), lambda i: (0, 0)),
            pl.BlockSpec((E * R, D), lambda i: (0, 0)),
        ],
        out_specs=pl.BlockSpec((_BT, D), lambda i: (i, 0)),
        out_shape=jax.ShapeDtypeStruct((T, D), jnp.float32),
    )(h, w_t, rep, b_flat)
    return out


# speedup vs baseline: 1.2925x; 1.2925x over previous
import jax
import jax.numpy as jnp
from jax import lax
from jax.experimental import pallas as pl

_E = 8
_K = 2
_R = 8
_ALPHA = 32.0
_SCALING = _ALPHA / _R

_BT = 2048


def _moe_body(x_ref, wg_ref, bg_ref, af_ref, bf_ref, rep_ref, o_ref):
    xv = x_ref[...]
    logits = jnp.dot(xv, wg_ref[...], preferred_element_type=jnp.float32)
    logits_t = logits.T + bg_ref[...]

    row = lax.broadcasted_iota(jnp.int32, logits_t.shape, 0).astype(jnp.float32)
    neg_inf = jnp.float32(-jnp.inf)
    big = jnp.float32(_E)

    m1 = jnp.max(logits_t, axis=0, keepdims=True)
    i1 = jnp.min(jnp.where(logits_t == m1, row, big), axis=0, keepdims=True)
    sel1 = row == i1
    l2 = jnp.where(sel1, neg_inf, logits_t)
    m2 = jnp.max(l2, axis=0, keepdims=True)
    i2 = jnp.min(jnp.where(l2 == m2, row, big), axis=0, keepdims=True)
    sel2 = row == i2

    p2 = jnp.exp(m2 - m1)
    w1 = 1.0 / (1.0 + p2)
    w2 = 1.0 - w1
    zero = jnp.float32(0.0)
    w_t = jnp.where(sel1, w1, zero) + jnp.where(sel2, w2, zero)

    h = jnp.dot(
        xv.astype(jnp.bfloat16), af_ref[...], preferred_element_type=jnp.float32
    )
    w_rep = lax.dot_general(
        w_t,
        rep_ref[...],
        dimension_numbers=(((0,), (0,)), ((), ())),
        preferred_element_type=jnp.float32,
    )
    g = (h * w_rep).astype(jnp.bfloat16)
    o_ref[...] = jnp.dot(g, bf_ref[...], preferred_element_type=jnp.float32)


@jax.jit
def kernel(x, Wg, bg, A, B):
    T, D = x.shape
    E, R, _ = A.shape
    a_flat = A.reshape(E * R, D).T.astype(jnp.bfloat16)
    b_flat = (
        (B.transpose(0, 2, 1) * jnp.float32(_SCALING))
        .reshape(E * R, D)
        .astype(jnp.bfloat16)
    )
    rep = jnp.repeat(jnp.eye(E, dtype=jnp.float32), R, axis=1)
    bg2 = bg.reshape(E, 1)

    grid = (T // _BT,)
    return pl.pallas_call(
        _moe_body,
        grid=grid,
        in_specs=[
            pl.BlockSpec((_BT, D), lambda i: (i, 0)),
            pl.BlockSpec((D, E), lambda i: (0, 0)),
            pl.BlockSpec((E, 1), lambda i: (0, 0)),
            pl.BlockSpec((D, E * R), lambda i: (0, 0)),
            pl.BlockSpec((E * R, D), lambda i: (0, 0)),
            pl.BlockSpec((E, E * R), lambda i: (0, 0)),
        ],
        out_specs=pl.BlockSpec((_BT, D), lambda i: (i, 0)),
        out_shape=jax.ShapeDtypeStruct((T, D), jnp.float32),
    )(x, Wg.T, bg2, a_flat, b_flat, rep)
